# baseline (speedup 1.0000x reference)
import jax
import jax.numpy as jnp
from jax.experimental import pallas as pl
from jax.experimental.pallas import tpu as pltpu

N = 10000
NFEAT = 128
NHID = 16
NCLASS = 7
BR = 400
NB = N // BR


def _body(adj_ref, x_ref, pk_ref, out_ref, s1_scr, s2_scr):
    g = pl.program_id(0)
    i = jax.lax.rem(g, NB)

    @pl.when(g == 0)
    def _():
        s1_scr[...] = jnp.dot(x_ref[...], pk_ref[0:128, :],
                              preferred_element_type=jnp.float32)

    @pl.when(g < NB)
    def _():
        h = jnp.dot(adj_ref[0], s1_scr[...],
                    preferred_element_type=jnp.float32)
        h = jnp.maximum(h + pk_ref[128:129, :], 0.0)
        s2_scr[pl.ds(i * BR, BR), :] = jnp.dot(
            h, pk_ref[136:152, 0:7], preferred_element_type=jnp.float32)

    @pl.when(g >= NB)
    def _():
        h2 = jnp.dot(adj_ref[0], s2_scr[...],
                     preferred_element_type=jnp.float32) + pk_ref[152:153, 0:7]
        o = jnp.dot(h2, pk_ref[160:167, 0:7],
                    preferred_element_type=jnp.float32) + pk_ref[168:169, 0:7]
        m = jnp.max(o, axis=-1, keepdims=True)
        e = o - m
        out_ref[pl.ds(i * BR, BR), :] = e - jnp.log(
            jnp.sum(jnp.exp(e), axis=-1, keepdims=True))


def kernel(x, adj, W1, b1, W2, b2, WL, bL):
    pk = (jnp.zeros((176, NHID), jnp.float32)
          .at[0:128, :].set(W1)
          .at[128, :].set(b1)
          .at[136:152, 0:NCLASS].set(W2)
          .at[152, 0:NCLASS].set(b2)
          .at[160:167, 0:NCLASS].set(WL)
          .at[168, 0:NCLASS].set(bL))
    c = lambda g: (0, 0)
    return pl.pallas_call(
        _body,
        grid=(2 * NB,),
        in_specs=[
            pl.BlockSpec((1, BR, N), lambda g: (g // NB, g % NB, 0)),
            pl.BlockSpec((N, NFEAT), c),
            pl.BlockSpec((176, NHID), c),
        ],
        out_specs=pl.BlockSpec((N, NCLASS), c),
        out_shape=jax.ShapeDtypeStruct((N, NCLASS), jnp.float32),
        scratch_shapes=[
            pltpu.VMEM((N, NHID), jnp.float32),
            pltpu.VMEM((N, NCLASS), jnp.float32),
        ],
    )(adj, x, pk)


# speedup vs baseline: 1.0152x; 1.0152x over previous
import jax
import jax.numpy as jnp
from jax.experimental import pallas as pl
from jax.experimental.pallas import tpu as pltpu

N = 10000
NFEAT = 128
NHID = 16
NCLASS = 7
BR = 400
NB = N // BR


def _body(adj_ref, x_ref, w1_ref, b1_ref, w2_ref, b2_ref, wl_ref, bl_ref,
          out_ref, s1_scr, s2_scr):
    g = pl.program_id(0)
    i = jax.lax.rem(g, NB)

    @pl.when(g == 0)
    def _():
        s1_scr[...] = jnp.dot(x_ref[...], w1_ref[...],
                              preferred_element_type=jnp.float32)

    @pl.when(g < NB)
    def _():
        h = jnp.dot(adj_ref[0].astype(jnp.bfloat16),
                    s1_scr[...].astype(jnp.bfloat16),
                    preferred_element_type=jnp.float32)
        h = jnp.maximum(h + b1_ref[...], 0.0)
        s2_scr[pl.ds(i * BR, BR), :] = jnp.dot(
            h, w2_ref[...], preferred_element_type=jnp.float32)

    @pl.when(g >= NB)
    def _():
        h2 = jnp.dot(adj_ref[0].astype(jnp.bfloat16),
                     s2_scr[...].astype(jnp.bfloat16),
                     preferred_element_type=jnp.float32) + b2_ref[...]
        o = jnp.dot(h2, wl_ref[...],
                    preferred_element_type=jnp.float32) + bl_ref[...]
        m = jnp.max(o, axis=-1, keepdims=True)
        e = o - m
        out_ref[pl.ds(i * BR, BR), :] = e - jnp.log(
            jnp.sum(jnp.exp(e), axis=-1, keepdims=True))


def kernel(x, adj, W1, b1, W2, b2, WL, bL):
    b1r = b1.reshape(1, NHID)
    b2r = b2.reshape(1, NCLASS)
    bLr = bL.reshape(1, NCLASS)
    c = lambda g: (0, 0)
    return pl.pallas_call(
        _body,
        grid=(2 * NB,),
        in_specs=[
            pl.BlockSpec((1, BR, N), lambda g: (g // NB, g % NB, 0)),
            pl.BlockSpec((N, NFEAT), c),
            pl.BlockSpec((NFEAT, NHID), c),
            pl.BlockSpec((1, NHID), c),
            pl.BlockSpec((NHID, NCLASS), c),
            pl.BlockSpec((1, NCLASS), c),
            pl.BlockSpec((NCLASS, NCLASS), c),
            pl.BlockSpec((1, NCLASS), c),
        ],
        out_specs=pl.BlockSpec((N, NCLASS), c),
        out_shape=jax.ShapeDtypeStruct((N, NCLASS), jnp.float32),
        scratch_shapes=[
            pltpu.VMEM((N, NHID), jnp.float32),
            pltpu.VMEM((N, NCLASS), jnp.float32),
        ],
    )(adj, x, W1, b1r, W2, b2r, WL, bLr)
